# baseline (speedup 1.0000x reference)
import functools

import jax
import jax.numpy as jnp
from jax.experimental import pallas as pl
from jax.experimental.pallas import tpu as pltpu

_N = 20000
_TOPK = 200
_G = 100
_ROWS = 160
_LANES = 128
_PADN = _ROWS * _LANES
_HI_KEY = 0x3F800000
THR = 0.3


def _body(l0, l1, r0, r1, r2, r3, a0, a1, a2, a3, y_sm, out_ref):
    f32 = jnp.float32
    HIGH = jax.lax.Precision.HIGHEST

    l0v = l0[...]
    l1v = l1[...]

    m = jnp.maximum(l0v, l1v)
    e0 = jnp.exp(l0v - m)
    e1 = jnp.exp(l1v - m)
    ssum = e0 + e1
    p0 = e0 / ssum
    p1 = e1 / ssum
    fg = p1 > p0
    s = jnp.where(fg, jnp.maximum(p0, p1), jnp.float32(-1.0))

    ridx = jax.lax.broadcasted_iota(jnp.int32, (_ROWS, _LANES), 0)
    lidx = jax.lax.broadcasted_iota(jnp.int32, (_ROWS, _LANES), 1)
    flat = ridx * _LANES + lidx
    valid = flat < _N
    s = jnp.where(valid, s, jnp.float32(-2.0))

    key = jnp.where(
        s > 0,
        jax.lax.bitcast_convert_type(s, jnp.int32),
        jnp.where(s == -1.0, jnp.int32(-1), jnp.int32(-2)),
    )

    def bs_step(_, lohi):
        lo, hi = lohi
        mid = lo + (hi - lo) // 2
        cnt = jnp.sum((key >= mid).astype(jnp.int32))
        big = cnt >= _TOPK
        return (jnp.where(big, mid, lo), jnp.where(big, hi, mid))

    cnt_pos = jnp.sum((key >= 0x3F000000).astype(jnp.int32))
    lo, hi = jax.lax.fori_loop(
        0, 24, bs_step, (jnp.int32(0x3F000000 - 1), jnp.int32(_HI_KEY))
    )
    v_key = jnp.where(cnt_pos >= _TOPK, lo, jnp.int32(-1))

    gt = key > v_key
    tie = key == v_key
    cnt_gt = jnp.sum(gt.astype(jnp.int32))
    needed = _TOPK - cnt_gt

    t_f = tie.astype(f32)
    iu0 = jax.lax.broadcasted_iota(jnp.int32, (_LANES, _LANES), 0)
    iu1 = jax.lax.broadcasted_iota(jnp.int32, (_LANES, _LANES), 1)
    upper = (iu0 <= iu1).astype(f32)
    il0 = jax.lax.broadcasted_iota(jnp.int32, (_ROWS, _ROWS), 0)
    il1 = jax.lax.broadcasted_iota(jnp.int32, (_ROWS, _ROWS), 1)
    lower = (il0 > il1).astype(f32)
    ones_l = jnp.ones((_LANES, _LANES), f32)
    rowpart = jax.lax.dot(lower, t_f, precision=HIGH)
    rowoff = jax.lax.dot(rowpart, ones_l, precision=HIGH)
    intrarow = jax.lax.dot(t_f, upper, precision=HIGH)
    prefix = rowoff + intrarow
    sel = gt | (tie & (prefix <= needed.astype(f32)))

    x = r0[...] * a2[...] + a0[...]
    yy = r1[...] * a3[...] + a1[...]
    w = jnp.exp(r2[...]) * a2[...]
    h = jnp.exp(r3[...]) * a3[...]
    xe = x + w
    ye = yy + h
    area = w * h

    td_chunks = []
    for c in range(_ROWS // 8):
        r0_, r1_ = c * 8, c * 8 + 8
        xs = jax.lax.slice(x, (r0_, 0), (r1_, _LANES))
        ys = jax.lax.slice(yy, (r0_, 0), (r1_, _LANES))
        xes = jax.lax.slice(xe, (r0_, 0), (r1_, _LANES))
        yes_ = jax.lax.slice(ye, (r0_, 0), (r1_, _LANES))
        ars = jax.lax.slice(area, (r0_, 0), (r1_, _LANES))

        def iou_step(g, td8):
            gx = y_sm[g, 0]
            gy = y_sm[g, 1]
            gw = y_sm[g, 2]
            gh = y_sm[g, 3]
            dw = jnp.minimum(xes, gx + gw) - jnp.maximum(xs, gx)
            dh = jnp.minimum(yes_, gy + gh) - jnp.maximum(ys, gy)
            inter = dw * dh
            u = (ars + gw * gh) - inter
            hit = (inter > jnp.float32(THR) * u) ^ (u < 0)
            return td8 | hit.astype(jnp.int32)

        td_chunks.append(
            jax.lax.fori_loop(
                0, _G, iou_step, jnp.zeros((8, _LANES), jnp.int32),
                unroll=10,
            )
        )
    td = jnp.concatenate(td_chunks, axis=0) != 0

    term = jnp.where(td, jnp.log(1.0 - s), jnp.log(s))
    loss = jnp.sum(jnp.where(sel, term, jnp.float32(0.0)))
    out_ref[0, 0] = loss


def kernel(reg_preds, cls_preds, anchors, y):
    pad = _PADN - _N

    def col(arr, c):
        return jnp.pad(arr[:, c], (0, pad)).reshape(_ROWS, _LANES)

    ins = (
        col(cls_preds, 0), col(cls_preds, 1),
        col(reg_preds, 0), col(reg_preds, 1), col(reg_preds, 2), col(reg_preds, 3),
        col(anchors, 0), col(anchors, 1), col(anchors, 2), col(anchors, 3),
        y,
    )
    vspec = pl.BlockSpec(memory_space=pltpu.VMEM)
    sspec = pl.BlockSpec(memory_space=pltpu.SMEM)
    out = pl.pallas_call(
        _body,
        out_shape=jax.ShapeDtypeStruct((1, 1), jnp.float32),
        in_specs=[vspec] * 10 + [sspec],
        out_specs=pl.BlockSpec(memory_space=pltpu.SMEM),
    )(*ins)
    return out[0, 0]


# speedup vs baseline: 1.1621x; 1.1621x over previous
import functools

import jax
import jax.numpy as jnp
from jax.experimental import pallas as pl
from jax.experimental.pallas import tpu as pltpu

_N = 20000
_TOPK = 200
_G = 100
_ROWS = 160
_LANES = 128
_PADN = _ROWS * _LANES
_HI_KEY = 0x3F800000
THR = 0.3


def _body(l0, l1, r0, r1, r2, r3, a0, a1, a2, a3, y_sm, out_ref):
    f32 = jnp.float32
    HIGH = jax.lax.Precision.HIGHEST

    l0v = l0[...]
    l1v = l1[...]

    m = jnp.maximum(l0v, l1v)
    e0 = jnp.exp(l0v - m)
    e1 = jnp.exp(l1v - m)
    ssum = e0 + e1
    p0 = e0 / ssum
    p1 = e1 / ssum
    fg = p1 > p0
    s = jnp.where(fg, jnp.maximum(p0, p1), jnp.float32(-1.0))

    ridx = jax.lax.broadcasted_iota(jnp.int32, (_ROWS, _LANES), 0)
    lidx = jax.lax.broadcasted_iota(jnp.int32, (_ROWS, _LANES), 1)
    flat = ridx * _LANES + lidx
    valid = flat < _N
    s = jnp.where(valid, s, jnp.float32(-2.0))

    key = jnp.where(
        s > 0,
        jax.lax.bitcast_convert_type(s, jnp.int32),
        jnp.where(s == -1.0, jnp.int32(-1), jnp.int32(-2)),
    )

    def bs_step(_, lohi):
        lo, hi = lohi
        mid = lo + (hi - lo) // 2
        cnt = jnp.sum((key >= mid).astype(jnp.int32))
        big = cnt >= _TOPK
        return (jnp.where(big, mid, lo), jnp.where(big, hi, mid))

    cnt_pos = jnp.sum((key >= 0x3F000000).astype(jnp.int32))
    lo, hi = jax.lax.fori_loop(
        0, 24, bs_step, (jnp.int32(0x3F000000 - 1), jnp.int32(_HI_KEY))
    )
    v_key = jnp.where(cnt_pos >= _TOPK, lo, jnp.int32(-1))

    gt = key > v_key
    tie = key == v_key
    cnt_gt = jnp.sum(gt.astype(jnp.int32))
    needed = _TOPK - cnt_gt

    t_f = tie.astype(f32)
    iu0 = jax.lax.broadcasted_iota(jnp.int32, (_LANES, _LANES), 0)
    iu1 = jax.lax.broadcasted_iota(jnp.int32, (_LANES, _LANES), 1)
    upper = (iu0 <= iu1).astype(f32)
    il0 = jax.lax.broadcasted_iota(jnp.int32, (_ROWS, _ROWS), 0)
    il1 = jax.lax.broadcasted_iota(jnp.int32, (_ROWS, _ROWS), 1)
    lower = (il0 > il1).astype(f32)
    ones_l = jnp.ones((_LANES, _LANES), f32)
    rowpart = jax.lax.dot(lower, t_f, precision=HIGH)
    rowoff = jax.lax.dot(rowpart, ones_l, precision=HIGH)
    intrarow = jax.lax.dot(t_f, upper, precision=HIGH)
    prefix = rowoff + intrarow
    sel = gt | (tie & (prefix <= needed.astype(f32)))

    x = r0[...] * a2[...] + a0[...]
    yy = r1[...] * a3[...] + a1[...]
    w = jnp.exp(r2[...]) * a2[...]
    h = jnp.exp(r3[...]) * a3[...]
    xe = x + w
    ye = yy + h
    area = w * h

    td_chunks = []
    _CR = 32
    for c in range(_ROWS // _CR):
        r0_, r1_ = c * _CR, c * _CR + _CR
        xs = jax.lax.slice(x, (r0_, 0), (r1_, _LANES))
        ys = jax.lax.slice(yy, (r0_, 0), (r1_, _LANES))
        xes = jax.lax.slice(xe, (r0_, 0), (r1_, _LANES))
        yes_ = jax.lax.slice(ye, (r0_, 0), (r1_, _LANES))
        ars = jax.lax.slice(area, (r0_, 0), (r1_, _LANES))

        def iou_step(g, td8):
            gx = y_sm[g, 0]
            gy = y_sm[g, 1]
            gw = y_sm[g, 2]
            gh = y_sm[g, 3]
            dw = jnp.minimum(xes, gx + gw) - jnp.maximum(xs, gx)
            dh = jnp.minimum(yes_, gy + gh) - jnp.maximum(ys, gy)
            inter = dw * dh
            u = (ars + gw * gh) - inter
            hit = (inter > jnp.float32(THR) * u) ^ (u < 0)
            return td8 | hit.astype(jnp.int32)

        td_chunks.append(
            jax.lax.fori_loop(
                0, _G, iou_step, jnp.zeros((_CR, _LANES), jnp.int32),
                unroll=4,
            )
        )
    td = jnp.concatenate(td_chunks, axis=0) != 0

    term = jnp.where(td, jnp.log(1.0 - s), jnp.log(s))
    loss = jnp.sum(jnp.where(sel, term, jnp.float32(0.0)))
    out_ref[0, 0] = loss


def kernel(reg_preds, cls_preds, anchors, y):
    pad = _PADN - _N

    def col(arr, c):
        return jnp.pad(arr[:, c], (0, pad)).reshape(_ROWS, _LANES)

    ins = (
        col(cls_preds, 0), col(cls_preds, 1),
        col(reg_preds, 0), col(reg_preds, 1), col(reg_preds, 2), col(reg_preds, 3),
        col(anchors, 0), col(anchors, 1), col(anchors, 2), col(anchors, 3),
        y,
    )
    vspec = pl.BlockSpec(memory_space=pltpu.VMEM)
    sspec = pl.BlockSpec(memory_space=pltpu.SMEM)
    out = pl.pallas_call(
        _body,
        out_shape=jax.ShapeDtypeStruct((1, 1), jnp.float32),
        in_specs=[vspec] * 10 + [sspec],
        out_specs=pl.BlockSpec(memory_space=pltpu.SMEM),
    )(*ins)
    return out[0, 0]


# speedup vs baseline: 1.2219x; 1.0515x over previous
import functools

import jax
import jax.numpy as jnp
from jax.experimental import pallas as pl
from jax.experimental.pallas import tpu as pltpu

_N = 20000
_TOPK = 200
_G = 100
_ROWS = 160
_LANES = 128
_PADN = _ROWS * _LANES
_GPAD = 104
_HI_KEY = 0x3F800000
THR = 0.3


def _body(l0, l1, r0, r1, r2, r3, a0, a1, a2, a3, yv, out_ref,
          gx_s, gy_s, gxe_s, gye_s, ga_s):
    f32 = jnp.float32
    HIGH = jax.lax.Precision.HIGHEST

    l0v = l0[...]
    l1v = l1[...]

    m = jnp.maximum(l0v, l1v)
    e0 = jnp.exp(l0v - m)
    e1 = jnp.exp(l1v - m)
    ssum = e0 + e1
    p0 = e0 / ssum
    p1 = e1 / ssum
    fg = p1 > p0
    s = jnp.where(fg, jnp.maximum(p0, p1), jnp.float32(-1.0))

    ridx = jax.lax.broadcasted_iota(jnp.int32, (_ROWS, _LANES), 0)
    lidx = jax.lax.broadcasted_iota(jnp.int32, (_ROWS, _LANES), 1)
    flat = ridx * _LANES + lidx
    valid = flat < _N
    s = jnp.where(valid, s, jnp.float32(-2.0))

    key = jnp.where(
        s > 0,
        jax.lax.bitcast_convert_type(s, jnp.int32),
        jnp.where(s == -1.0, jnp.int32(-1), jnp.int32(-2)),
    )

    def bs_step(_, lohi):
        lo, hi = lohi
        mid = lo + (hi - lo) // 2
        cnt = jnp.sum((key >= mid).astype(jnp.int32))
        big = cnt >= _TOPK
        return (jnp.where(big, mid, lo), jnp.where(big, hi, mid))

    cnt_pos = jnp.sum((key >= 0x3F000000).astype(jnp.int32))
    lo, hi = jax.lax.fori_loop(
        0, 24, bs_step, (jnp.int32(0x3F000000 - 1), jnp.int32(_HI_KEY))
    )
    v_key = jnp.where(cnt_pos >= _TOPK, lo, jnp.int32(-1))

    gt = key > v_key
    tie = key == v_key
    cnt_gt = jnp.sum(gt.astype(jnp.int32))
    needed = _TOPK - cnt_gt

    t_f = tie.astype(f32)
    iu0 = jax.lax.broadcasted_iota(jnp.int32, (_LANES, _LANES), 0)
    iu1 = jax.lax.broadcasted_iota(jnp.int32, (_LANES, _LANES), 1)
    upper = (iu0 <= iu1).astype(f32)
    il0 = jax.lax.broadcasted_iota(jnp.int32, (_ROWS, _ROWS), 0)
    il1 = jax.lax.broadcasted_iota(jnp.int32, (_ROWS, _ROWS), 1)
    lower = (il0 > il1).astype(f32)
    ones_l = jnp.ones((_LANES, _LANES), f32)
    rowpart = jax.lax.dot(lower, t_f, precision=HIGH)
    rowoff = jax.lax.dot(rowpart, ones_l, precision=HIGH)
    intrarow = jax.lax.dot(t_f, upper, precision=HIGH)
    prefix = rowoff + intrarow
    sel = gt | (tie & (prefix <= needed.astype(f32)))

    x = r0[...] * a2[...] + a0[...]
    yy = r1[...] * a3[...] + a1[...]
    w = jnp.exp(r2[...]) * a2[...]
    h = jnp.exp(r3[...]) * a3[...]
    xe = x + w
    ye = yy + h
    area = w * h

    gx_c = yv[:, 0:1]
    gy_c = yv[:, 1:2]
    gw_c = yv[:, 2:3]
    gh_c = yv[:, 3:4]
    bshape = (_GPAD, _LANES)
    bc = lambda v: jnp.broadcast_to(v, bshape)
    gx_s[...] = bc(gx_c)
    gy_s[...] = bc(gy_c)
    gxe_s[...] = bc(gx_c + gw_c)
    gye_s[...] = bc(gy_c + gh_c)
    ga_s[...] = bc(gw_c * gh_c)

    td_chunks = []
    _CR = 32
    for c in range(_ROWS // _CR):
        r0_, r1_ = c * _CR, c * _CR + _CR
        xs = jax.lax.slice(x, (r0_, 0), (r1_, _LANES))
        ys = jax.lax.slice(yy, (r0_, 0), (r1_, _LANES))
        xes = jax.lax.slice(xe, (r0_, 0), (r1_, _LANES))
        yes_ = jax.lax.slice(ye, (r0_, 0), (r1_, _LANES))
        ars = jax.lax.slice(area, (r0_, 0), (r1_, _LANES))

        def iou_step(g, td8):
            gxr = gx_s[pl.ds(g, 1), :]
            gyr = gy_s[pl.ds(g, 1), :]
            gxer = gxe_s[pl.ds(g, 1), :]
            gyer = gye_s[pl.ds(g, 1), :]
            gar = ga_s[pl.ds(g, 1), :]
            dw = jnp.minimum(xes, gxer) - jnp.maximum(xs, gxr)
            dh = jnp.minimum(yes_, gyer) - jnp.maximum(ys, gyr)
            inter = dw * dh
            u = (ars + gar) - inter
            hit = (inter > jnp.float32(THR) * u) ^ (u < 0)
            return td8 | hit.astype(jnp.int32)

        td_chunks.append(
            jax.lax.fori_loop(
                0, _G, iou_step, jnp.zeros((_CR, _LANES), jnp.int32),
                unroll=4,
            )
        )
    td = jnp.concatenate(td_chunks, axis=0) != 0

    term = jnp.where(td, jnp.log(1.0 - s), jnp.log(s))
    loss = jnp.sum(jnp.where(sel, term, jnp.float32(0.0)))
    out_ref[0, 0] = loss


def kernel(reg_preds, cls_preds, anchors, y):
    pad = _PADN - _N

    def col(arr, c):
        return jnp.pad(arr[:, c], (0, pad)).reshape(_ROWS, _LANES)

    ins = (
        col(cls_preds, 0), col(cls_preds, 1),
        col(reg_preds, 0), col(reg_preds, 1), col(reg_preds, 2), col(reg_preds, 3),
        col(anchors, 0), col(anchors, 1), col(anchors, 2), col(anchors, 3),
        jnp.pad(y, ((0, _GPAD - _G), (0, _LANES - 4))),
    )
    vspec = pl.BlockSpec(memory_space=pltpu.VMEM)
    out = pl.pallas_call(
        _body,
        out_shape=jax.ShapeDtypeStruct((1, 1), jnp.float32),
        in_specs=[vspec] * 11,
        out_specs=pl.BlockSpec(memory_space=pltpu.SMEM),
        scratch_shapes=[pltpu.VMEM((_GPAD, _LANES), jnp.float32)] * 5,
    )(*ins)
    return out[0, 0]


# speedup vs baseline: 1.4193x; 1.1616x over previous
import functools

import jax
import jax.numpy as jnp
from jax.experimental import pallas as pl
from jax.experimental.pallas import tpu as pltpu

_N = 20000
_TOPK = 200
_G = 100
_ROWS = 160
_LANES = 128
_PADN = _ROWS * _LANES
_GPAD = 104
_HI_KEY = 0x3F800000
THR = 0.3


def _body(big, yv, out_ref, gx_s, gy_s, gxe_s, gye_s, ga_s):
    l0 = big[0]
    l1 = big[1]
    r0 = big[2]
    r1 = big[3]
    r2 = big[4]
    r3 = big[5]
    a0 = big[6]
    a1 = big[7]
    a2 = big[8]
    a3 = big[9]
    f32 = jnp.float32
    HIGH = jax.lax.Precision.HIGHEST

    l0v = l0
    l1v = l1

    m = jnp.maximum(l0v, l1v)
    e0 = jnp.exp(l0v - m)
    e1 = jnp.exp(l1v - m)
    ssum = e0 + e1
    p0 = e0 / ssum
    p1 = e1 / ssum
    fg = p1 > p0
    s = jnp.where(fg, jnp.maximum(p0, p1), jnp.float32(-1.0))

    ridx = jax.lax.broadcasted_iota(jnp.int32, (_ROWS, _LANES), 0)
    lidx = jax.lax.broadcasted_iota(jnp.int32, (_ROWS, _LANES), 1)
    flat = ridx * _LANES + lidx
    valid = flat < _N
    s = jnp.where(valid, s, jnp.float32(-2.0))

    key = jnp.where(
        s > 0,
        jax.lax.bitcast_convert_type(s, jnp.int32),
        jnp.where(s == -1.0, jnp.int32(-1), jnp.int32(-2)),
    )

    def bs_step(_, lohi):
        lo, hi = lohi
        mid = lo + (hi - lo) // 2
        cnt = jnp.sum((key >= mid).astype(jnp.int32))
        big = cnt >= _TOPK
        return (jnp.where(big, mid, lo), jnp.where(big, hi, mid))

    cnt_pos = jnp.sum((key >= 0x3F000000).astype(jnp.int32))
    lo, hi = jax.lax.fori_loop(
        0, 24, bs_step, (jnp.int32(0x3F000000 - 1), jnp.int32(_HI_KEY))
    )
    v_key = jnp.where(cnt_pos >= _TOPK, lo, jnp.int32(-1))

    gt = key > v_key
    tie = key == v_key
    cnt_gt = jnp.sum(gt.astype(jnp.int32))
    needed = _TOPK - cnt_gt

    t_f = tie.astype(f32)
    iu0 = jax.lax.broadcasted_iota(jnp.int32, (_LANES, _LANES), 0)
    iu1 = jax.lax.broadcasted_iota(jnp.int32, (_LANES, _LANES), 1)
    upper = (iu0 <= iu1).astype(f32)
    il0 = jax.lax.broadcasted_iota(jnp.int32, (_ROWS, _ROWS), 0)
    il1 = jax.lax.broadcasted_iota(jnp.int32, (_ROWS, _ROWS), 1)
    lower = (il0 > il1).astype(f32)
    ones_l = jnp.ones((_LANES, _LANES), f32)
    rowpart = jax.lax.dot(lower, t_f, precision=HIGH)
    rowoff = jax.lax.dot(rowpart, ones_l, precision=HIGH)
    intrarow = jax.lax.dot(t_f, upper, precision=HIGH)
    prefix = rowoff + intrarow
    sel = gt | (tie & (prefix <= needed.astype(f32)))

    x = r0 * a2 + a0
    yy = r1 * a3 + a1
    w = jnp.exp(r2) * a2
    h = jnp.exp(r3) * a3
    xe = x + w
    ye = yy + h
    area = w * h

    gx_c = yv[:, 0:1]
    gy_c = yv[:, 1:2]
    gw_c = yv[:, 2:3]
    gh_c = yv[:, 3:4]
    bshape = (_GPAD, _LANES)
    bc = lambda v: jnp.broadcast_to(v, bshape)
    gx_s[...] = bc(gx_c)
    gy_s[...] = bc(gy_c)
    gxe_s[...] = bc(gx_c + gw_c)
    gye_s[...] = bc(gy_c + gh_c)
    ga_s[...] = bc(gw_c * gh_c)

    td_chunks = []
    _CR = 32
    for c in range(_ROWS // _CR):
        r0_, r1_ = c * _CR, c * _CR + _CR
        xs = jax.lax.slice(x, (r0_, 0), (r1_, _LANES))
        ys = jax.lax.slice(yy, (r0_, 0), (r1_, _LANES))
        xes = jax.lax.slice(xe, (r0_, 0), (r1_, _LANES))
        yes_ = jax.lax.slice(ye, (r0_, 0), (r1_, _LANES))
        ars = jax.lax.slice(area, (r0_, 0), (r1_, _LANES))

        def iou_step(g, td8):
            gxr = gx_s[pl.ds(g, 1), :]
            gyr = gy_s[pl.ds(g, 1), :]
            gxer = gxe_s[pl.ds(g, 1), :]
            gyer = gye_s[pl.ds(g, 1), :]
            gar = ga_s[pl.ds(g, 1), :]
            dw = jnp.minimum(xes, gxer) - jnp.maximum(xs, gxr)
            dh = jnp.minimum(yes_, gyer) - jnp.maximum(ys, gyr)
            inter = dw * dh
            u = (ars + gar) - inter
            hit = (inter > jnp.float32(THR) * u) ^ (u < 0)
            return td8 | hit.astype(jnp.int32)

        td_chunks.append(
            jax.lax.fori_loop(
                0, _G, iou_step, jnp.zeros((_CR, _LANES), jnp.int32),
                unroll=4,
            )
        )
    td = jnp.concatenate(td_chunks, axis=0) != 0

    term = jnp.where(td, jnp.log(1.0 - s), jnp.log(s))
    loss = jnp.sum(jnp.where(sel, term, jnp.float32(0.0)))
    out_ref[0, 0] = loss


def kernel(reg_preds, cls_preds, anchors, y):
    pad = _PADN - _N

    def col(arr, c):
        return jnp.pad(arr[:, c], (0, pad)).reshape(_ROWS, _LANES)

    big = jnp.stack([
        col(cls_preds, 0), col(cls_preds, 1),
        col(reg_preds, 0), col(reg_preds, 1), col(reg_preds, 2), col(reg_preds, 3),
        col(anchors, 0), col(anchors, 1), col(anchors, 2), col(anchors, 3),
    ])
    ins = (big, jnp.pad(y, ((0, _GPAD - _G), (0, _LANES - 4))))
    vspec = pl.BlockSpec(memory_space=pltpu.VMEM)
    out = pl.pallas_call(
        _body,
        out_shape=jax.ShapeDtypeStruct((1, 1), jnp.float32),
        in_specs=[vspec] * 2,
        out_specs=pl.BlockSpec(memory_space=pltpu.SMEM),
        scratch_shapes=[pltpu.VMEM((_GPAD, _LANES), jnp.float32)] * 5,
    )(*ins)
    return out[0, 0]


# speedup vs baseline: 1.5879x; 1.1188x over previous
import functools

import jax
import jax.numpy as jnp
from jax.experimental import pallas as pl
from jax.experimental.pallas import tpu as pltpu

_N = 20000
_TOPK = 200
_G = 100
_ROWS = 160
_LANES = 128
_PADN = _ROWS * _LANES
_GPAD = 104
_HI_KEY = 0x3F800000
THR = 0.3


def _body(big, yv, out_ref, gx_s, gy_s, gxe_s, gye_s, ga_s):
    l0 = big[0]
    l1 = big[1]
    r0 = big[2]
    r1 = big[3]
    r2 = big[4]
    r3 = big[5]
    a0 = big[6]
    a1 = big[7]
    a2 = big[8]
    a3 = big[9]
    f32 = jnp.float32
    HIGH = jax.lax.Precision.HIGHEST

    l0v = l0
    l1v = l1

    m = jnp.maximum(l0v, l1v)
    e0 = jnp.exp(l0v - m)
    e1 = jnp.exp(l1v - m)
    ssum = e0 + e1
    p0 = e0 / ssum
    p1 = e1 / ssum
    fg = p1 > p0
    s = jnp.where(fg, jnp.maximum(p0, p1), jnp.float32(-1.0))

    ridx = jax.lax.broadcasted_iota(jnp.int32, (_ROWS, _LANES), 0)
    lidx = jax.lax.broadcasted_iota(jnp.int32, (_ROWS, _LANES), 1)
    flat = ridx * _LANES + lidx
    valid = flat < _N
    s = jnp.where(valid, s, jnp.float32(-2.0))

    key = jnp.where(
        s > 0,
        jax.lax.bitcast_convert_type(s, jnp.int32),
        jnp.where(s == -1.0, jnp.int32(-1), jnp.int32(-2)),
    )

    def bs_round(_, lohi):
        lo, hi = lohi
        step = (hi - lo + 7) // 8
        ge = [
            (jnp.sum((key >= (lo + j * step)).astype(jnp.int32)) >= _TOPK)
            .astype(jnp.int32)
            for j in range(1, 8)
        ]
        jbest = sum(ge)
        new_lo = lo + jbest * step
        new_hi = jnp.minimum(lo + (jbest + 1) * step, hi)
        return new_lo, new_hi

    cnt_pos = jnp.sum((key >= 0x3F000000).astype(jnp.int32))
    lo, hi = jax.lax.fori_loop(
        0, 8, bs_round, (jnp.int32(0x3F000000 - 1), jnp.int32(_HI_KEY))
    )
    v_key = jnp.where(cnt_pos >= _TOPK, lo, jnp.int32(-1))

    gt = key > v_key
    tie = key == v_key
    cnt_gt = jnp.sum(gt.astype(jnp.int32))
    needed = _TOPK - cnt_gt

    t_f = tie.astype(f32)
    iu0 = jax.lax.broadcasted_iota(jnp.int32, (_LANES, _LANES), 0)
    iu1 = jax.lax.broadcasted_iota(jnp.int32, (_LANES, _LANES), 1)
    upper = (iu0 <= iu1).astype(f32)
    il0 = jax.lax.broadcasted_iota(jnp.int32, (_ROWS, _ROWS), 0)
    il1 = jax.lax.broadcasted_iota(jnp.int32, (_ROWS, _ROWS), 1)
    lower = (il0 > il1).astype(f32)
    ones_l = jnp.ones((_LANES, _LANES), f32)
    rowpart = jax.lax.dot(lower, t_f, precision=HIGH)
    rowoff = jax.lax.dot(rowpart, ones_l, precision=HIGH)
    intrarow = jax.lax.dot(t_f, upper, precision=HIGH)
    prefix = rowoff + intrarow
    sel = gt | (tie & (prefix <= needed.astype(f32)))

    x = r0 * a2 + a0
    yy = r1 * a3 + a1
    w = jnp.exp(r2) * a2
    h = jnp.exp(r3) * a3
    xe = x + w
    ye = yy + h
    area = w * h

    gx_c = yv[:, 0:1]
    gy_c = yv[:, 1:2]
    gw_c = yv[:, 2:3]
    gh_c = yv[:, 3:4]
    bshape = (_GPAD, _LANES)
    bc = lambda v: jnp.broadcast_to(v, bshape)
    gx_s[...] = bc(gx_c)
    gy_s[...] = bc(gy_c)
    gxe_s[...] = bc(gx_c + gw_c)
    gye_s[...] = bc(gy_c + gh_c)
    ga_s[...] = bc(gw_c * gh_c)

    td_chunks = []
    _CR = 32
    for c in range(_ROWS // _CR):
        r0_, r1_ = c * _CR, c * _CR + _CR
        xs = jax.lax.slice(x, (r0_, 0), (r1_, _LANES))
        ys = jax.lax.slice(yy, (r0_, 0), (r1_, _LANES))
        xes = jax.lax.slice(xe, (r0_, 0), (r1_, _LANES))
        yes_ = jax.lax.slice(ye, (r0_, 0), (r1_, _LANES))
        ars = jax.lax.slice(area, (r0_, 0), (r1_, _LANES))

        def iou_step(g, td8):
            gxr = gx_s[pl.ds(g, 1), :]
            gyr = gy_s[pl.ds(g, 1), :]
            gxer = gxe_s[pl.ds(g, 1), :]
            gyer = gye_s[pl.ds(g, 1), :]
            gar = ga_s[pl.ds(g, 1), :]
            dw = jnp.minimum(xes, gxer) - jnp.maximum(xs, gxr)
            dh = jnp.minimum(yes_, gyer) - jnp.maximum(ys, gyr)
            inter = dw * dh
            u = (ars + gar) - inter
            hit = (inter > jnp.float32(THR) * u) ^ (u < 0)
            return td8 | hit.astype(jnp.int32)

        td_chunks.append(
            jax.lax.fori_loop(
                0, _G, iou_step, jnp.zeros((_CR, _LANES), jnp.int32),
                unroll=4,
            )
        )
    td = jnp.concatenate(td_chunks, axis=0) != 0

    term = jnp.where(td, jnp.log(1.0 - s), jnp.log(s))
    loss = jnp.sum(jnp.where(sel, term, jnp.float32(0.0)))
    out_ref[0, 0] = loss


def kernel(reg_preds, cls_preds, anchors, y):
    pad = _PADN - _N

    def col(arr, c):
        return jnp.pad(arr[:, c], (0, pad)).reshape(_ROWS, _LANES)

    big = jnp.stack([
        col(cls_preds, 0), col(cls_preds, 1),
        col(reg_preds, 0), col(reg_preds, 1), col(reg_preds, 2), col(reg_preds, 3),
        col(anchors, 0), col(anchors, 1), col(anchors, 2), col(anchors, 3),
    ])
    ins = (big, jnp.pad(y, ((0, _GPAD - _G), (0, _LANES - 4))))
    vspec = pl.BlockSpec(memory_space=pltpu.VMEM)
    out = pl.pallas_call(
        _body,
        out_shape=jax.ShapeDtypeStruct((1, 1), jnp.float32),
        in_specs=[vspec] * 2,
        out_specs=pl.BlockSpec(memory_space=pltpu.SMEM),
        scratch_shapes=[pltpu.VMEM((_GPAD, _LANES), jnp.float32)] * 5,
    )(*ins)
    return out[0, 0]


# speedup vs baseline: 1.5933x; 1.0034x over previous
import functools

import jax
import jax.numpy as jnp
from jax.experimental import pallas as pl
from jax.experimental.pallas import tpu as pltpu

_N = 20000
_TOPK = 200
_G = 100
_ROWS = 160
_LANES = 128
_PADN = _ROWS * _LANES
_GPAD = 104
_HI_KEY = 0x3F800000
THR = 0.3


def _body(big, yv, out_ref, gx_s, gy_s, gxe_s, gye_s, ga_s):
    l0 = big[0]
    l1 = big[1]
    r0 = big[2]
    r1 = big[3]
    r2 = big[4]
    r3 = big[5]
    a0 = big[6]
    a1 = big[7]
    a2 = big[8]
    a3 = big[9]
    f32 = jnp.float32
    HIGH = jax.lax.Precision.HIGHEST

    l0v = l0
    l1v = l1

    m = jnp.maximum(l0v, l1v)
    e0 = jnp.exp(l0v - m)
    e1 = jnp.exp(l1v - m)
    ssum = e0 + e1
    p0 = e0 / ssum
    p1 = e1 / ssum
    fg = p1 > p0
    s = jnp.where(fg, jnp.maximum(p0, p1), jnp.float32(-1.0))

    ridx = jax.lax.broadcasted_iota(jnp.int32, (_ROWS, _LANES), 0)
    lidx = jax.lax.broadcasted_iota(jnp.int32, (_ROWS, _LANES), 1)
    flat = ridx * _LANES + lidx
    valid = flat < _N
    s = jnp.where(valid, s, jnp.float32(-2.0))

    key = jnp.where(
        s > 0,
        jax.lax.bitcast_convert_type(s, jnp.int32),
        jnp.where(s == -1.0, jnp.int32(-1), jnp.int32(-2)),
    )

    def bs_round(_, lohi):
        lo, hi = lohi
        step = (hi - lo + 7) // 8
        ge = [
            (jnp.sum((key >= (lo + j * step)).astype(jnp.int32)) >= _TOPK)
            .astype(jnp.int32)
            for j in range(1, 8)
        ]
        jbest = sum(ge)
        new_lo = lo + jbest * step
        new_hi = jnp.minimum(lo + (jbest + 1) * step, hi)
        return new_lo, new_hi

    cnt_pos = jnp.sum((key >= 0x3F000000).astype(jnp.int32))
    lo, hi = jax.lax.fori_loop(
        0, 8, bs_round, (jnp.int32(0x3F000000 - 1), jnp.int32(_HI_KEY))
    )
    v_key = jnp.where(cnt_pos >= _TOPK, lo, jnp.int32(-1))

    gt = key > v_key
    tie = key == v_key
    cnt_gt = jnp.sum(gt.astype(jnp.int32))
    needed = _TOPK - cnt_gt

    t_f = tie.astype(f32)
    iu0 = jax.lax.broadcasted_iota(jnp.int32, (_LANES, _LANES), 0)
    iu1 = jax.lax.broadcasted_iota(jnp.int32, (_LANES, _LANES), 1)
    upper = (iu0 <= iu1).astype(f32)
    il0 = jax.lax.broadcasted_iota(jnp.int32, (_ROWS, _ROWS), 0)
    il1 = jax.lax.broadcasted_iota(jnp.int32, (_ROWS, _ROWS), 1)
    lower = (il0 > il1).astype(f32)
    ones_l = jnp.ones((_LANES, _LANES), f32)
    rowpart = jax.lax.dot(lower, t_f, precision=HIGH)
    rowoff = jax.lax.dot(rowpart, ones_l, precision=HIGH)
    intrarow = jax.lax.dot(t_f, upper, precision=HIGH)
    prefix = rowoff + intrarow
    sel = gt | (tie & (prefix <= needed.astype(f32)))

    x = r0 * a2 + a0
    yy = r1 * a3 + a1
    w = jnp.exp(r2) * a2
    h = jnp.exp(r3) * a3
    xe = x + w
    ye = yy + h
    area = w * h

    gx_c = yv[:, 0:1]
    gy_c = yv[:, 1:2]
    gw_c = yv[:, 2:3]
    gh_c = yv[:, 3:4]
    bshape = (_GPAD, _LANES)
    bc = lambda v: jnp.broadcast_to(v, bshape)
    gx_s[...] = bc(gx_c)
    gy_s[...] = bc(gy_c)
    gxe_s[...] = bc(gx_c + gw_c)
    gye_s[...] = bc(gy_c + gh_c)
    ga_s[...] = bc(gw_c * gh_c)

    td_chunks = []
    _CR = 40
    for c in range(_ROWS // _CR):
        r0_, r1_ = c * _CR, c * _CR + _CR
        xs = jax.lax.slice(x, (r0_, 0), (r1_, _LANES))
        ys = jax.lax.slice(yy, (r0_, 0), (r1_, _LANES))
        xes = jax.lax.slice(xe, (r0_, 0), (r1_, _LANES))
        yes_ = jax.lax.slice(ye, (r0_, 0), (r1_, _LANES))
        ars = jax.lax.slice(area, (r0_, 0), (r1_, _LANES))

        def iou_step(g, td8):
            gxr = gx_s[pl.ds(g, 1), :]
            gyr = gy_s[pl.ds(g, 1), :]
            gxer = gxe_s[pl.ds(g, 1), :]
            gyer = gye_s[pl.ds(g, 1), :]
            gar = ga_s[pl.ds(g, 1), :]
            dw = jnp.minimum(xes, gxer) - jnp.maximum(xs, gxr)
            dh = jnp.minimum(yes_, gyer) - jnp.maximum(ys, gyr)
            inter = dw * dh
            u = (ars + gar) - inter
            hit = (inter > jnp.float32(THR) * u) ^ (u < 0)
            return td8 | hit.astype(jnp.int32)

        td_chunks.append(
            jax.lax.fori_loop(
                0, _G, iou_step, jnp.zeros((_CR, _LANES), jnp.int32),
                unroll=5,
            )
        )
    td = jnp.concatenate(td_chunks, axis=0) != 0

    term = jnp.where(td, jnp.log(1.0 - s), jnp.log(s))
    loss = jnp.sum(jnp.where(sel, term, jnp.float32(0.0)))
    out_ref[0, 0] = loss


def kernel(reg_preds, cls_preds, anchors, y):
    pad = _PADN - _N

    def col(arr, c):
        return jnp.pad(arr[:, c], (0, pad)).reshape(_ROWS, _LANES)

    big = jnp.stack([
        col(cls_preds, 0), col(cls_preds, 1),
        col(reg_preds, 0), col(reg_preds, 1), col(reg_preds, 2), col(reg_preds, 3),
        col(anchors, 0), col(anchors, 1), col(anchors, 2), col(anchors, 3),
    ])
    ins = (big, jnp.pad(y, ((0, _GPAD - _G), (0, _LANES - 4))))
    vspec = pl.BlockSpec(memory_space=pltpu.VMEM)
    out = pl.pallas_call(
        _body,
        out_shape=jax.ShapeDtypeStruct((1, 1), jnp.float32),
        in_specs=[vspec] * 2,
        out_specs=pl.BlockSpec(memory_space=pltpu.SMEM),
        scratch_shapes=[pltpu.VMEM((_GPAD, _LANES), jnp.float32)] * 5,
    )(*ins)
    return out[0, 0]


# speedup vs baseline: 1.5943x; 1.0007x over previous
import functools

import jax
import jax.numpy as jnp
from jax.experimental import pallas as pl
from jax.experimental.pallas import tpu as pltpu

_N = 20000
_TOPK = 200
_G = 100
_ROWS = 160
_LANES = 128
_PADN = _ROWS * _LANES
_GPAD = 104
_HI_KEY = 0x3F800000
THR = 0.3


def _body(big, yv, out_ref, gx_s, gy_s, gxe_s, gye_s, ga_s):
    l0 = big[0]
    l1 = big[1]
    r0 = big[2]
    r1 = big[3]
    r2 = big[4]
    r3 = big[5]
    a0 = big[6]
    a1 = big[7]
    a2 = big[8]
    a3 = big[9]
    f32 = jnp.float32
    HIGH = jax.lax.Precision.HIGHEST

    l0v = l0
    l1v = l1

    m = jnp.maximum(l0v, l1v)
    e0 = jnp.exp(l0v - m)
    e1 = jnp.exp(l1v - m)
    ssum = e0 + e1
    p0 = e0 / ssum
    p1 = e1 / ssum
    fg = p1 > p0
    s = jnp.where(fg, jnp.maximum(p0, p1), jnp.float32(-1.0))

    ridx = jax.lax.broadcasted_iota(jnp.int32, (_ROWS, _LANES), 0)
    lidx = jax.lax.broadcasted_iota(jnp.int32, (_ROWS, _LANES), 1)
    flat = ridx * _LANES + lidx
    valid = flat < _N
    s = jnp.where(valid, s, jnp.float32(-2.0))

    key = jnp.where(
        s > 0,
        jax.lax.bitcast_convert_type(s, jnp.int32),
        jnp.where(s == -1.0, jnp.int32(-1), jnp.int32(-2)),
    )

    def bs_round(_, lohi):
        lo, hi = lohi
        step = (hi - lo + 15) // 16
        ge = [
            (jnp.sum((key >= (lo + j * step)).astype(jnp.int32)) >= _TOPK)
            .astype(jnp.int32)
            for j in range(1, 16)
        ]
        jbest = sum(ge)
        new_lo = lo + jbest * step
        new_hi = jnp.minimum(lo + (jbest + 1) * step, hi)
        return new_lo, new_hi

    cnt_pos = jnp.sum((key >= 0x3F000000).astype(jnp.int32))
    lo, hi = jax.lax.fori_loop(
        0, 6, bs_round, (jnp.int32(0x3F000000 - 1), jnp.int32(_HI_KEY))
    )
    v_key = jnp.where(cnt_pos >= _TOPK, lo, jnp.int32(-1))

    gt = key > v_key
    tie = key == v_key
    cnt_gt = jnp.sum(gt.astype(jnp.int32))
    needed = _TOPK - cnt_gt

    t_f = tie.astype(f32)
    iu0 = jax.lax.broadcasted_iota(jnp.int32, (_LANES, _LANES), 0)
    iu1 = jax.lax.broadcasted_iota(jnp.int32, (_LANES, _LANES), 1)
    upper = (iu0 <= iu1).astype(f32)
    il0 = jax.lax.broadcasted_iota(jnp.int32, (_ROWS, _ROWS), 0)
    il1 = jax.lax.broadcasted_iota(jnp.int32, (_ROWS, _ROWS), 1)
    lower = (il0 > il1).astype(f32)
    ones_l = jnp.ones((_LANES, _LANES), f32)
    rowpart = jax.lax.dot(lower, t_f, precision=HIGH)
    rowoff = jax.lax.dot(rowpart, ones_l, precision=HIGH)
    intrarow = jax.lax.dot(t_f, upper, precision=HIGH)
    prefix = rowoff + intrarow
    sel = gt | (tie & (prefix <= needed.astype(f32)))

    x = r0 * a2 + a0
    yy = r1 * a3 + a1
    w = jnp.exp(r2) * a2
    h = jnp.exp(r3) * a3
    xe = x + w
    ye = yy + h
    area = w * h

    gx_c = yv[:, 0:1]
    gy_c = yv[:, 1:2]
    gw_c = yv[:, 2:3]
    gh_c = yv[:, 3:4]
    bshape = (_GPAD, _LANES)
    bc = lambda v: jnp.broadcast_to(v, bshape)
    gx_s[...] = bc(gx_c)
    gy_s[...] = bc(gy_c)
    gxe_s[...] = bc(gx_c + gw_c)
    gye_s[...] = bc(gy_c + gh_c)
    ga_s[...] = bc(gw_c * gh_c)

    td_chunks = []
    _CR = 40
    for c in range(_ROWS // _CR):
        r0_, r1_ = c * _CR, c * _CR + _CR
        xs = jax.lax.slice(x, (r0_, 0), (r1_, _LANES))
        ys = jax.lax.slice(yy, (r0_, 0), (r1_, _LANES))
        xes = jax.lax.slice(xe, (r0_, 0), (r1_, _LANES))
        yes_ = jax.lax.slice(ye, (r0_, 0), (r1_, _LANES))
        ars = jax.lax.slice(area, (r0_, 0), (r1_, _LANES))

        def iou_step(g, td8):
            gxr = gx_s[pl.ds(g, 1), :]
            gyr = gy_s[pl.ds(g, 1), :]
            gxer = gxe_s[pl.ds(g, 1), :]
            gyer = gye_s[pl.ds(g, 1), :]
            gar = ga_s[pl.ds(g, 1), :]
            dw = jnp.minimum(xes, gxer) - jnp.maximum(xs, gxr)
            dh = jnp.minimum(yes_, gyer) - jnp.maximum(ys, gyr)
            inter = dw * dh
            u = (ars + gar) - inter
            hit = (inter > jnp.float32(THR) * u) ^ (u < 0)
            return td8 | hit.astype(jnp.int32)

        td_chunks.append(
            jax.lax.fori_loop(
                0, _G, iou_step, jnp.zeros((_CR, _LANES), jnp.int32),
                unroll=5,
            )
        )
    td = jnp.concatenate(td_chunks, axis=0) != 0

    term = jnp.where(td, jnp.log(1.0 - s), jnp.log(s))
    loss = jnp.sum(jnp.where(sel, term, jnp.float32(0.0)))
    out_ref[0, 0] = loss


def kernel(reg_preds, cls_preds, anchors, y):
    pad = _PADN - _N

    def col(arr, c):
        return jnp.pad(arr[:, c], (0, pad)).reshape(_ROWS, _LANES)

    big = jnp.stack([
        col(cls_preds, 0), col(cls_preds, 1),
        col(reg_preds, 0), col(reg_preds, 1), col(reg_preds, 2), col(reg_preds, 3),
        col(anchors, 0), col(anchors, 1), col(anchors, 2), col(anchors, 3),
    ])
    ins = (big, jnp.pad(y, ((0, _GPAD - _G), (0, _LANES - 4))))
    vspec = pl.BlockSpec(memory_space=pltpu.VMEM)
    out = pl.pallas_call(
        _body,
        out_shape=jax.ShapeDtypeStruct((1, 1), jnp.float32),
        in_specs=[vspec] * 2,
        out_specs=pl.BlockSpec(memory_space=pltpu.SMEM),
        scratch_shapes=[pltpu.VMEM((_GPAD, _LANES), jnp.float32)] * 5,
    )(*ins)
    return out[0, 0]


# speedup vs baseline: 1.5963x; 1.0013x over previous
import jax
import jax.numpy as jnp
from jax.experimental import pallas as pl
from jax.experimental.pallas import tpu as pltpu

_N = 20000
_TOPK = 200
_G = 100
_ROWS = 160
_LANES = 128
_PADN = _ROWS * _LANES
_GPAD = 104
_HI_KEY = 0x3F800000
THR = 0.3


def _body(big, yv, out_ref, gx_s, gy_s, gxe_s, gye_s, ga_s):
    l0 = big[0]
    l1 = big[1]
    r0 = big[2]
    r1 = big[3]
    r2 = big[4]
    r3 = big[5]
    a0 = big[6]
    a1 = big[7]
    a2 = big[8]
    a3 = big[9]
    f32 = jnp.float32
    HIGH = jax.lax.Precision.HIGHEST

    l0v = l0
    l1v = l1

    m = jnp.maximum(l0v, l1v)
    e0 = jnp.exp(l0v - m)
    e1 = jnp.exp(l1v - m)
    ssum = e0 + e1
    p0 = e0 / ssum
    p1 = e1 / ssum
    fg = p1 > p0
    s = jnp.where(fg, jnp.maximum(p0, p1), jnp.float32(-1.0))

    ridx = jax.lax.broadcasted_iota(jnp.int32, (_ROWS, _LANES), 0)
    lidx = jax.lax.broadcasted_iota(jnp.int32, (_ROWS, _LANES), 1)
    flat = ridx * _LANES + lidx
    valid = flat < _N
    s = jnp.where(valid, s, jnp.float32(-2.0))

    key = jnp.where(
        s > 0,
        jax.lax.bitcast_convert_type(s, jnp.int32),
        jnp.where(s == -1.0, jnp.int32(-1), jnp.int32(-2)),
    )

    def bs_round(_, lohi):
        lo, hi = lohi
        step = (hi - lo + 15) // 16
        ge = [
            (jnp.sum((key >= (lo + j * step)).astype(jnp.int32)) >= _TOPK)
            .astype(jnp.int32)
            for j in range(1, 16)
        ]
        jbest = sum(ge)
        new_lo = lo + jbest * step
        new_hi = jnp.minimum(lo + (jbest + 1) * step, hi)
        return new_lo, new_hi

    cnt_pos = jnp.sum((key >= 0x3F000000).astype(jnp.int32))
    lo, hi = jax.lax.fori_loop(
        0, 6, bs_round, (jnp.int32(0x3F000000 - 1), jnp.int32(_HI_KEY))
    )
    v_key = jnp.where(cnt_pos >= _TOPK, lo, jnp.int32(-1))

    gt = key > v_key
    tie = key == v_key
    cnt_gt = jnp.sum(gt.astype(jnp.int32))
    needed = _TOPK - cnt_gt

    t_f = tie.astype(f32)
    iu0 = jax.lax.broadcasted_iota(jnp.int32, (_LANES, _LANES), 0)
    iu1 = jax.lax.broadcasted_iota(jnp.int32, (_LANES, _LANES), 1)
    upper = (iu0 <= iu1).astype(f32)
    il0 = jax.lax.broadcasted_iota(jnp.int32, (_ROWS, _ROWS), 0)
    il1 = jax.lax.broadcasted_iota(jnp.int32, (_ROWS, _ROWS), 1)
    lower = (il0 > il1).astype(f32)
    ones_l = jnp.ones((_LANES, _LANES), f32)
    rowpart = jax.lax.dot(lower, t_f, precision=HIGH)
    rowoff = jax.lax.dot(rowpart, ones_l, precision=HIGH)
    intrarow = jax.lax.dot(t_f, upper, precision=HIGH)
    prefix = rowoff + intrarow
    sel = gt | (tie & (prefix <= needed.astype(f32)))

    x = r0 * a2 + a0
    yy = r1 * a3 + a1
    w = jnp.exp(r2) * a2
    h = jnp.exp(r3) * a3
    xe = x + w
    ye = yy + h
    area = w * h

    gx_c = yv[:, 0:1]
    gy_c = yv[:, 1:2]
    gw_c = yv[:, 2:3]
    gh_c = yv[:, 3:4]
    bshape = (_GPAD, _LANES)
    bc = lambda v: jnp.broadcast_to(v, bshape)
    gx_s[...] = bc(gx_c)
    gy_s[...] = bc(gy_c)
    gxe_s[...] = bc(gx_c + gw_c)
    gye_s[...] = bc(gy_c + gh_c)
    ga_s[...] = bc(gw_c * gh_c)

    td_chunks = []
    _CR = 40
    for c in range(_ROWS // _CR):
        r0_, r1_ = c * _CR, c * _CR + _CR
        xs = jax.lax.slice(x, (r0_, 0), (r1_, _LANES))
        ys = jax.lax.slice(yy, (r0_, 0), (r1_, _LANES))
        xes = jax.lax.slice(xe, (r0_, 0), (r1_, _LANES))
        yes_ = jax.lax.slice(ye, (r0_, 0), (r1_, _LANES))
        ars = jax.lax.slice(area, (r0_, 0), (r1_, _LANES))

        def iou_step(g, td8):
            gxr = gx_s[pl.ds(g, 1), :]
            gyr = gy_s[pl.ds(g, 1), :]
            gxer = gxe_s[pl.ds(g, 1), :]
            gyer = gye_s[pl.ds(g, 1), :]
            gar = ga_s[pl.ds(g, 1), :]
            dw = jnp.minimum(xes, gxer) - jnp.maximum(xs, gxr)
            dh = jnp.minimum(yes_, gyer) - jnp.maximum(ys, gyr)
            inter = dw * dh
            u = (ars + gar) - inter
            hit = (inter > jnp.float32(THR) * u) ^ (u < 0)
            return td8 | hit.astype(jnp.int32)

        td_chunks.append(
            jax.lax.fori_loop(
                0, _G, iou_step, jnp.zeros((_CR, _LANES), jnp.int32),
                unroll=5,
            )
        )
    td = jnp.concatenate(td_chunks, axis=0) != 0

    term = jnp.where(td, jnp.log(1.0 - s), jnp.log(s))
    loss = jnp.sum(jnp.where(sel, term, jnp.float32(0.0)))
    out_ref[0, 0] = loss


def kernel(reg_preds, cls_preds, anchors, y):
    pad = _PADN - _N

    def col(arr, c):
        return jnp.pad(arr[:, c], (0, pad)).reshape(_ROWS, _LANES)

    big = jnp.stack([
        col(cls_preds, 0), col(cls_preds, 1),
        col(reg_preds, 0), col(reg_preds, 1), col(reg_preds, 2), col(reg_preds, 3),
        col(anchors, 0), col(anchors, 1), col(anchors, 2), col(anchors, 3),
    ])
    ins = (big, jnp.pad(y, ((0, _GPAD - _G), (0, _LANES - 4))))
    vspec = pl.BlockSpec(memory_space=pltpu.VMEM)
    out = pl.pallas_call(
        _body,
        out_shape=jax.ShapeDtypeStruct((1, 1), jnp.float32),
        in_specs=[vspec] * 2,
        out_specs=pl.BlockSpec(memory_space=pltpu.SMEM),
        scratch_shapes=[pltpu.VMEM((_GPAD, _LANES), jnp.float32)] * 5,
    )(*ins)
    return out[0, 0]
